# Initial kernel scaffold; baseline (speedup 1.0000x reference)
import jax
import jax.numpy as jnp
from jax.experimental import pallas as pl


def kernel(x, W_feat, W_dir, bn_w, bn_b):
    raise NotImplementedError("write your pallas kernel here")


# speedup vs baseline: 7.0825x; 7.0825x over previous
import functools

import jax
import jax.numpy as jnp
from jax import lax
from jax.experimental import pallas as pl
from jax.experimental.pallas import tpu as pltpu
from jax.experimental.pallas import tpu_sc as plsc

KP1 = 21
EPS = 1e-06
BN_EPS = 1e-05
RB = 256


def _knn_body(xft_ref, xf_ref, idx_ref, cnt_ref):
    b = pl.program_id(0)
    rb = pl.program_id(1)
    xft = xft_ref[0]
    xf = xf_ref[0]
    n = xf.shape[1]
    xx_row = jnp.sum(xf * xf, axis=0, keepdims=True)
    xx_blk = jnp.sum(xft * xft, axis=1, keepdims=True)
    prod = lax.dot_general(
        xft, xf, (((1,), (0,)), ((), ())),
                           preferred_element_type=jnp.float32)
    dist = 2.0 * prod - xx_blk - xx_row
    lane = lax.broadcasted_iota(jnp.int32, (RB, n), 1)
    neg = jnp.float32(-jnp.inf)
    for k in range(KP1):
        m = jnp.max(dist, axis=1, keepdims=True)
        amax = jnp.min(jnp.where(dist == m, lane, n), axis=1)
        idx_ref[0, k, :] = amax + b * n
        dist = jnp.where(lane == amax[:, None], neg, dist)
    sel = (dist == neg).astype(jnp.float32)

    @pl.when(rb == 0)
    def _():
        cnt_ref[0] = jnp.zeros_like(cnt_ref[0])

    cnt_ref[0] += jnp.sum(sel, axis=0, keepdims=True)


def _point_body(xfl_ref, wf_ref, wd_ref, bnw_ref, bnb_ref, cnt_ref, g_ref):
    b_sz = xfl_ref.shape[0]
    n = cnt_ref.shape[2]
    wf = wf_ref[...]
    wd = wd_ref[...]
    ps, ds, nrms = [], [], []
    s1 = jnp.zeros((wf.shape[0], 1), jnp.float32)
    s2 = jnp.zeros((wf.shape[0], 1), jnp.float32)
    for b in range(b_sz):
        xb = xfl_ref[b]
        p = lax.dot_general(wf, xb, (((1,), (0,)), ((), ())),
                            preferred_element_type=jnp.float32)
        d = lax.dot_general(wd, xb, (((1,), (0,)), ((), ())),
                            preferred_element_type=jnp.float32)
        nsq = (p[:, 0:n] * p[:, 0:n] + p[:, n:2 * n] * p[:, n:2 * n]
               + p[:, 2 * n:] * p[:, 2 * n:])
        nrm = jnp.sqrt(nsq)
        cb = cnt_ref[b]
        s1 = s1 + jnp.sum(nrm * cb, axis=1, keepdims=True)
        s2 = s2 + jnp.sum(nsq * cb, axis=1, keepdims=True)
        ps.append(p)
        ds.append(d)
        nrms.append(nrm)
    denom = jnp.float32(1.0 / (b_sz * n * KP1))
    mean = s1 * denom
    var = s2 * denom - mean * mean
    rstd = lax.rsqrt(var + BN_EPS)
    bnw = bnw_ref[...]
    bnb = bnb_ref[...]
    for b in range(b_sz):
        p, d, nrm = ps[b], ds[b], nrms[b]
        norm_bn = (nrm - mean) * rstd * bnw + bnb
        sc = norm_bn / nrm
        gts = []
        dot = jnp.zeros((wf.shape[0], n), jnp.float32)
        dsq = jnp.zeros((wf.shape[0], n), jnp.float32)
        phs, dts = [], []
        for t in range(3):
            pt = p[:, t * n:(t + 1) * n] * sc
            dt = d[:, t * n:(t + 1) * n]
            dot = dot + pt * dt
            dsq = dsq + dt * dt
            phs.append(pt)
            dts.append(dt)
        coef = jnp.where(dot >= 0, 0.0, dot / (dsq + EPS))
        inv = jnp.float32(1.0 / KP1)
        for t in range(3):
            gts.append((phs[t] - coef * dts[t]) * inv)
        gmat = jnp.concatenate(gts, axis=0)
        pad = jnp.zeros((n, g_ref.shape[2] - gmat.shape[0]), jnp.float32)
        g_ref[b] = jnp.concatenate([gmat.T, pad], axis=-1)


SUB = 128


def _make_gather(total_rows, d, workers):
    ch = total_rows // workers
    nsub = ch // SUB
    mesh = plsc.VectorSubcoreMesh(core_axis_name="c", subcore_axis_name="s")

    @functools.partial(
        pl.kernel, mesh=mesh,
        out_type=jax.ShapeDtypeStruct((total_rows, d), jnp.float32),
        scratch_types=[
            pltpu.VMEM((KP1 * nsub * SUB,), jnp.int32),
            pltpu.VMEM((ch, d), jnp.float32),
        ],
    )
    def gather_k(g_hbm, idxr_hbm, out_hbm, idx_v, acc_v):
        nc = lax.axis_size("c")
        wid = lax.axis_index("s") * nc + lax.axis_index("c")
        pltpu.sync_copy(idxr_hbm.at[wid], idx_v)
        for s in range(nsub):
            pltpu.sync_copy(g_hbm.at[idx_v.at[pl.ds(s * SUB, SUB)]],
                            acc_v.at[pl.ds(s * SUB, SUB)])

        def body(k, carry):
            for s in range(nsub):
                pltpu.sync_copy(
                    g_hbm.at[idx_v.at[pl.ds((k * nsub + s) * SUB, SUB)]],
                    acc_v.at[pl.ds(s * SUB, SUB)], add=True)
            return carry

        lax.fori_loop(1, KP1, body, 0)
        pltpu.sync_copy(acc_v, out_hbm.at[pl.ds(wid * ch, ch)])

    return gather_k


def kernel(x, W_feat, W_dir, bn_w, bn_b):
    B, C, _, N = x.shape
    Cout = W_feat.shape[0]
    F = C * 3
    xf = x.reshape(B, F, N)
    xft = jnp.swapaxes(xf, 1, 2)

    idx, cnt = pl.pallas_call(
        _knn_body,
        grid=(B, N // RB),
        in_specs=[
            pl.BlockSpec((1, RB, F), lambda b, r: (b, r, 0)),
            pl.BlockSpec((1, F, N), lambda b, r: (b, 0, 0)),
        ],
        out_specs=[
            pl.BlockSpec((1, KP1, RB), lambda b, r: (b, 0, r)),
            pl.BlockSpec((1, 1, N), lambda b, r: (b, 0, 0)),
        ],
        out_shape=[
            jax.ShapeDtypeStruct((B, KP1, N), jnp.int32),
            jax.ShapeDtypeStruct((B, 1, N), jnp.float32),
        ],
    )(xft, xf)

    g = pl.pallas_call(
        _point_body,
        out_shape=jax.ShapeDtypeStruct((B, N, SUB), jnp.float32),
    )(x.reshape(B, C, 3 * N), W_feat, W_dir,
      bn_w.reshape(Cout, 1), bn_b.reshape(Cout, 1), cnt)

    workers = 32
    ch = B * N // workers
    nsub = ch // SUB
    cpb = N // ch
    idxr = (idx.reshape(B, KP1, cpb, nsub, SUB)
            .transpose(0, 2, 1, 3, 4)
            .reshape(workers, KP1 * nsub * SUB))
    gather_k = _make_gather(B * N, SUB, workers)
    out_flat = gather_k(g.reshape(B * N, SUB), idxr)
    return (out_flat[:, :3 * Cout]
            .reshape(B, N, 3, Cout).transpose(0, 3, 2, 1))


# speedup vs baseline: 8.6415x; 1.2201x over previous
import functools

import jax
import jax.numpy as jnp
from jax import lax
from jax.experimental import pallas as pl
from jax.experimental.pallas import tpu as pltpu
from jax.experimental.pallas import tpu_sc as plsc

KP1 = 21
EPS = 1e-06
BN_EPS = 1e-05
RB = 256


def _knn_body(xft_ref, xf_ref, idx_ref, cnt_ref):
    b = pl.program_id(0)
    rb = pl.program_id(1)
    xft = xft_ref[0]
    xf = xf_ref[0]
    n = xf.shape[1]
    xx_row = jnp.sum(xf * xf, axis=0, keepdims=True)
    xx_blk = jnp.sum(xft * xft, axis=1, keepdims=True)
    prod = lax.dot_general(
        xft, xf, (((1,), (0,)), ((), ())),
                           preferred_element_type=jnp.float32)
    dist = 2.0 * prod - xx_blk - xx_row
    lane = lax.broadcasted_iota(jnp.int32, (RB, n), 1)
    neg = jnp.float32(-jnp.inf)
    for k in range(KP1):
        amax = jnp.argmax(dist, axis=1).astype(jnp.int32)
        idx_ref[0, k, :] = amax + b * n
        dist = jnp.where(lane == amax[:, None], neg, dist)
    sel = (dist == neg).astype(jnp.float32)

    @pl.when(rb == 0)
    def _():
        cnt_ref[0] = jnp.zeros_like(cnt_ref[0])

    cnt_ref[0] += jnp.sum(sel, axis=0, keepdims=True)


def _point_body(xfl_ref, wf_ref, wd_ref, bnw_ref, bnb_ref, cnt_ref, g_ref):
    b_sz = xfl_ref.shape[0]
    n = cnt_ref.shape[2]
    wf = wf_ref[...]
    wd = wd_ref[...]
    ps, ds, nrms = [], [], []
    s1 = jnp.zeros((wf.shape[0], 1), jnp.float32)
    s2 = jnp.zeros((wf.shape[0], 1), jnp.float32)
    for b in range(b_sz):
        xb = xfl_ref[b]
        p = lax.dot_general(wf, xb, (((1,), (0,)), ((), ())),
                            preferred_element_type=jnp.float32)
        d = lax.dot_general(wd, xb, (((1,), (0,)), ((), ())),
                            preferred_element_type=jnp.float32)
        nsq = (p[:, 0:n] * p[:, 0:n] + p[:, n:2 * n] * p[:, n:2 * n]
               + p[:, 2 * n:] * p[:, 2 * n:])
        nrm = jnp.sqrt(nsq)
        cb = cnt_ref[b]
        s1 = s1 + jnp.sum(nrm * cb, axis=1, keepdims=True)
        s2 = s2 + jnp.sum(nsq * cb, axis=1, keepdims=True)
        ps.append(p)
        ds.append(d)
        nrms.append(nrm)
    denom = jnp.float32(1.0 / (b_sz * n * KP1))
    mean = s1 * denom
    var = s2 * denom - mean * mean
    rstd = lax.rsqrt(var + BN_EPS)
    bnw = bnw_ref[...]
    bnb = bnb_ref[...]
    for b in range(b_sz):
        p, d, nrm = ps[b], ds[b], nrms[b]
        norm_bn = (nrm - mean) * rstd * bnw + bnb
        sc = norm_bn / nrm
        gts = []
        dot = jnp.zeros((wf.shape[0], n), jnp.float32)
        dsq = jnp.zeros((wf.shape[0], n), jnp.float32)
        phs, dts = [], []
        for t in range(3):
            pt = p[:, t * n:(t + 1) * n] * sc
            dt = d[:, t * n:(t + 1) * n]
            dot = dot + pt * dt
            dsq = dsq + dt * dt
            phs.append(pt)
            dts.append(dt)
        coef = jnp.where(dot >= 0, 0.0, dot / (dsq + EPS))
        inv = jnp.float32(1.0 / KP1)
        for t in range(3):
            gts.append((phs[t] - coef * dts[t]) * inv)
        gmat = jnp.concatenate(gts, axis=0)
        pad = jnp.zeros((n, g_ref.shape[2] - gmat.shape[0]), jnp.float32)
        g_ref[b] = jnp.concatenate([gmat.T, pad], axis=-1)


SUB = 128


def _make_gather(total_rows, d, workers):
    ch = total_rows // workers
    nsub = ch // SUB
    mesh = plsc.VectorSubcoreMesh(core_axis_name="c", subcore_axis_name="s")

    @functools.partial(
        pl.kernel, mesh=mesh,
        out_type=jax.ShapeDtypeStruct((total_rows, d), jnp.float32),
        scratch_types=[
            pltpu.VMEM((KP1 * nsub * SUB,), jnp.int32),
            pltpu.VMEM((ch, d), jnp.float32),
        ],
    )
    def gather_k(g_hbm, idxr_hbm, out_hbm, idx_v, acc_v):
        nc = lax.axis_size("c")
        wid = lax.axis_index("s") * nc + lax.axis_index("c")
        pltpu.sync_copy(idxr_hbm.at[wid], idx_v)
        for s in range(nsub):
            pltpu.sync_copy(g_hbm.at[idx_v.at[pl.ds(s * SUB, SUB)]],
                            acc_v.at[pl.ds(s * SUB, SUB)])

        def body(k, carry):
            for s in range(nsub):
                pltpu.sync_copy(
                    g_hbm.at[idx_v.at[pl.ds((k * nsub + s) * SUB, SUB)]],
                    acc_v.at[pl.ds(s * SUB, SUB)], add=True)
            return carry

        lax.fori_loop(1, KP1, body, 0)
        pltpu.sync_copy(acc_v, out_hbm.at[pl.ds(wid * ch, ch)])

    return gather_k


def kernel(x, W_feat, W_dir, bn_w, bn_b):
    B, C, _, N = x.shape
    Cout = W_feat.shape[0]
    F = C * 3
    xf = x.reshape(B, F, N)
    xft = jnp.swapaxes(xf, 1, 2)

    idx, cnt = pl.pallas_call(
        _knn_body,
        grid=(B, N // RB),
        in_specs=[
            pl.BlockSpec((1, RB, F), lambda b, r: (b, r, 0)),
            pl.BlockSpec((1, F, N), lambda b, r: (b, 0, 0)),
        ],
        out_specs=[
            pl.BlockSpec((1, KP1, RB), lambda b, r: (b, 0, r)),
            pl.BlockSpec((1, 1, N), lambda b, r: (b, 0, 0)),
        ],
        out_shape=[
            jax.ShapeDtypeStruct((B, KP1, N), jnp.int32),
            jax.ShapeDtypeStruct((B, 1, N), jnp.float32),
        ],
    )(xft, xf)

    g = pl.pallas_call(
        _point_body,
        out_shape=jax.ShapeDtypeStruct((B, N, SUB), jnp.float32),
    )(x.reshape(B, C, 3 * N), W_feat, W_dir,
      bn_w.reshape(Cout, 1), bn_b.reshape(Cout, 1), cnt)

    workers = 32
    ch = B * N // workers
    nsub = ch // SUB
    cpb = N // ch
    idxr = (idx.reshape(B, KP1, cpb, nsub, SUB)
            .transpose(0, 2, 1, 3, 4)
            .reshape(workers, KP1 * nsub * SUB))
    gather_k = _make_gather(B * N, SUB, workers)
    out_flat = gather_k(g.reshape(B * N, SUB), idxr)
    return (out_flat[:, :3 * Cout]
            .reshape(B, N, 3, Cout).transpose(0, 3, 2, 1))
